# initial kernel scaffold (unmeasured)
import functools

import jax
import jax.numpy as jnp
from jax import lax
from jax.experimental import pallas as pl
from jax.experimental.pallas import tpu as pltpu

N_DEV = 4
M_BLK = 1024
N_TOT = 8192
W = 1024
S = N_TOT // W


def _rs_body(p_ref, out_ref, amax_ref, comm_ref, send_sems, recv_sems):
    my = lax.axis_index("i")
    left = lax.rem(my + N_DEV - 1, N_DEV)
    right = lax.rem(my + 1, N_DEV)
    j = pl.program_id(0)

    barrier_sem = pltpu.get_barrier_semaphore()
    for nbr in (left, right):
        pl.semaphore_signal(
            barrier_sem, inc=1,
            device_id=(nbr,), device_id_type=pl.DeviceIdType.MESH,
        )
    pl.semaphore_wait(barrier_sem, 2)

    def p_chunk(c):
        return p_ref[pl.ds(c * M_BLK, M_BLK), :]

    comm_ref[0] = p_chunk(lax.rem(my + N_DEV - 1, N_DEV))

    for h in range(N_DEV - 1):
        send_slot = h % 2
        recv_slot = (h + 1) % 2
        rdma = pltpu.make_async_remote_copy(
            src_ref=comm_ref.at[send_slot],
            dst_ref=comm_ref.at[recv_slot],
            send_sem=send_sems.at[send_slot],
            recv_sem=recv_sems.at[recv_slot],
            device_id=(right,),
            device_id_type=pl.DeviceIdType.MESH,
        )
        rdma.start()
        rdma.wait()

        c = lax.rem(my + 2 * N_DEV - 2 - h, N_DEV)
        if h < N_DEV - 2:
            comm_ref[recv_slot] = comm_ref[recv_slot] + p_chunk(c)
        else:
            acc = comm_ref[recv_slot].astype(jnp.float32) + p_chunk(c).astype(
                jnp.float32
            )
            out_ref[...] = acc
            local_max = jnp.maximum(jnp.max(acc), 0.0)
            prev = jnp.where(j == 0, 0.0, amax_ref[0, 0])
            amax_ref[0, 0] = jnp.maximum(prev, local_max)


def _rs_call(partial):
    return pl.pallas_call(
        _rs_body,
        grid=(S,),
        in_specs=[
            pl.BlockSpec((N_DEV * M_BLK, W), lambda j: (0, j)),
        ],
        out_specs=[
            pl.BlockSpec((M_BLK, W), lambda j: (0, j)),
            pl.BlockSpec((1, 1), lambda j: (0, 0)),
        ],
        out_shape=[
            jax.ShapeDtypeStruct((M_BLK, N_TOT), jnp.float32),
            jax.ShapeDtypeStruct((1, 1), jnp.float32),
        ],
        scratch_shapes=[
            pltpu.VMEM((2, M_BLK, W), jnp.bfloat16),
            pltpu.SemaphoreType.DMA((2,)),
            pltpu.SemaphoreType.DMA((2,)),
        ],
        compiler_params=pltpu.CompilerParams(
            collective_id=0,
            dimension_semantics=("arbitrary",),
        ),
    )(partial)


def _amax_body(in_ref, out_ref, my_ref, comm_ref, send_sems, recv_sems):
    my = lax.axis_index("i")

    barrier_sem = pltpu.get_barrier_semaphore()
    for k in range(1, N_DEV):
        pl.semaphore_signal(
            barrier_sem, inc=1,
            device_id=(lax.rem(my + k, N_DEV),),
            device_id_type=pl.DeviceIdType.MESH,
        )
    pl.semaphore_wait(barrier_sem, N_DEV - 1)

    my_ref[...] = jnp.full((8, 128), in_ref[0, 0], jnp.float32)

    rdmas = []
    for k in range(1, N_DEV):
        target = lax.rem(my + k, N_DEV)
        rdma = pltpu.make_async_remote_copy(
            src_ref=my_ref,
            dst_ref=comm_ref.at[k],
            send_sem=send_sems.at[k],
            recv_sem=recv_sems.at[k],
            device_id=(target,),
            device_id_type=pl.DeviceIdType.MESH,
        )
        rdma.start()
        rdmas.append(rdma)
    for rdma in rdmas:
        rdma.wait()

    result = in_ref[0, 0]
    for k in range(1, N_DEV):
        result = jnp.maximum(result, comm_ref[k, 0, 0])
    out_ref[0, 0] = result


def _amax_call(amax_local):
    return pl.pallas_call(
        _amax_body,
        in_specs=[pl.BlockSpec(memory_space=pltpu.VMEM)],
        out_specs=pl.BlockSpec(memory_space=pltpu.VMEM),
        out_shape=jax.ShapeDtypeStruct((1, 1), jnp.float32),
        scratch_shapes=[
            pltpu.VMEM((8, 128), jnp.float32),
            pltpu.VMEM((N_DEV, 8, 128), jnp.float32),
            pltpu.SemaphoreType.DMA((N_DEV,)),
            pltpu.SemaphoreType.DMA((N_DEV,)),
        ],
        compiler_params=pltpu.CompilerParams(collective_id=1),
    )(amax_local)


def kernel(x, w_mat):
    partial = jnp.dot(
        x, w_mat, preferred_element_type=jnp.float32
    ).astype(jnp.bfloat16)

    y, amax_local = _rs_call(partial)
    amax = _amax_call(amax_local)[0, 0]

    y = jnp.maximum(y, 0.0)
    scale = amax / 448.0
    q = (y / scale).astype(jnp.float8_e4m3fn)
    return q.astype(jnp.float32) * scale


# baseline (device time: 717958 ns/iter reference)
import functools

import jax
import jax.numpy as jnp
from jax import lax
from jax.experimental import pallas as pl
from jax.experimental.pallas import tpu as pltpu

N_DEV = 4
M_BLK = 1024
N_TOT = 8192
W = 1024
S = N_TOT // W


def _rs_body(p_ref, out_ref, amax_ref, comm_ref, send_sems, recv_sems):
    my = lax.axis_index("i")
    left = lax.rem(my + N_DEV - 1, N_DEV)
    right = lax.rem(my + 1, N_DEV)
    j = pl.program_id(0)

    barrier_sem = pltpu.get_barrier_semaphore()
    for nbr in (left, right):
        pl.semaphore_signal(
            barrier_sem, inc=1,
            device_id=(nbr,), device_id_type=pl.DeviceIdType.MESH,
        )
    pl.semaphore_wait(barrier_sem, 2)

    def p_chunk(c):
        return p_ref[pl.ds(c * M_BLK, M_BLK), :]

    comm_ref[0] = p_chunk(lax.rem(my + N_DEV - 1, N_DEV))

    for h in range(N_DEV - 1):
        send_slot = h % 2
        recv_slot = (h + 1) % 2
        rdma = pltpu.make_async_remote_copy(
            src_ref=comm_ref.at[send_slot],
            dst_ref=comm_ref.at[recv_slot],
            send_sem=send_sems.at[send_slot],
            recv_sem=recv_sems.at[recv_slot],
            device_id=(right,),
            device_id_type=pl.DeviceIdType.MESH,
        )
        rdma.start()
        rdma.wait()

        c = lax.rem(my + 2 * N_DEV - 2 - h, N_DEV)
        if h < N_DEV - 2:
            comm_ref[recv_slot] = comm_ref[recv_slot] + p_chunk(c)
        else:
            acc = comm_ref[recv_slot].astype(jnp.float32) + p_chunk(c).astype(
                jnp.float32
            )
            out_ref[...] = acc
            local_max = jnp.full((8, 128), jnp.maximum(jnp.max(acc), 0.0))
            prev = jnp.where(j == 0, jnp.zeros((8, 128), jnp.float32),
                             amax_ref[...])
            amax_ref[...] = jnp.maximum(prev, local_max)


def _rs_call(partial):
    return pl.pallas_call(
        _rs_body,
        grid=(S,),
        in_specs=[
            pl.BlockSpec((N_DEV * M_BLK, W), lambda j: (0, j)),
        ],
        out_specs=[
            pl.BlockSpec((M_BLK, W), lambda j: (0, j)),
            pl.BlockSpec((8, 128), lambda j: (0, 0)),
        ],
        out_shape=[
            jax.ShapeDtypeStruct((M_BLK, N_TOT), jnp.float32),
            jax.ShapeDtypeStruct((8, 128), jnp.float32),
        ],
        scratch_shapes=[
            pltpu.VMEM((2, M_BLK, W), jnp.bfloat16),
            pltpu.SemaphoreType.DMA((2,)),
            pltpu.SemaphoreType.DMA((2,)),
        ],
        compiler_params=pltpu.CompilerParams(
            collective_id=0,
            dimension_semantics=("arbitrary",),
        ),
    )(partial)


def _amax_body(in_ref, out_ref, comm_ref, send_sems, recv_sems):
    my = lax.axis_index("i")

    barrier_sem = pltpu.get_barrier_semaphore()
    for k in range(1, N_DEV):
        pl.semaphore_signal(
            barrier_sem, inc=1,
            device_id=(lax.rem(my + k, N_DEV),),
            device_id_type=pl.DeviceIdType.MESH,
        )
    pl.semaphore_wait(barrier_sem, N_DEV - 1)

    rdmas = []
    for k in range(1, N_DEV):
        target = lax.rem(my + k, N_DEV)
        rdma = pltpu.make_async_remote_copy(
            src_ref=in_ref,
            dst_ref=comm_ref.at[k],
            send_sem=send_sems.at[k],
            recv_sem=recv_sems.at[k],
            device_id=(target,),
            device_id_type=pl.DeviceIdType.MESH,
        )
        rdma.start()
        rdmas.append(rdma)
    for rdma in rdmas:
        rdma.wait()

    result = in_ref[...]
    for k in range(1, N_DEV):
        result = jnp.maximum(result, comm_ref[k])
    out_ref[...] = result


def _amax_call(amax_local):
    return pl.pallas_call(
        _amax_body,
        in_specs=[pl.BlockSpec(memory_space=pltpu.VMEM)],
        out_specs=pl.BlockSpec(memory_space=pltpu.VMEM),
        out_shape=jax.ShapeDtypeStruct((8, 128), jnp.float32),
        scratch_shapes=[
            pltpu.VMEM((N_DEV, 8, 128), jnp.float32),
            pltpu.SemaphoreType.DMA((N_DEV,)),
            pltpu.SemaphoreType.DMA((N_DEV,)),
        ],
        compiler_params=pltpu.CompilerParams(collective_id=1),
    )(amax_local)


def kernel(x, w_mat):
    partial = jnp.dot(
        x, w_mat, preferred_element_type=jnp.float32
    ).astype(jnp.bfloat16)

    y, amax_local = _rs_call(partial)
    amax = _amax_call(amax_local)[0, 0]


    y = jnp.maximum(y, 0.0)
    scale = amax / 448.0
    q = (y / scale).astype(jnp.float8_e4m3fn)
    q = lax.optimization_barrier(q)
    return q.astype(jnp.float32) * scale


# device time: 445252 ns/iter; 1.6125x vs baseline; 1.6125x over previous
import functools

import jax
import jax.numpy as jnp
from jax import lax
from jax.experimental import pallas as pl
from jax.experimental.pallas import tpu as pltpu

N_DEV = 4
M_BLK = 1024
N_TOT = 8192
W = 1024
S = N_TOT // W


H = W // 2


def _rs_body(p_ref, out_ref, amax_ref,
             cw_ref, ccw_ref, cw_send, cw_recv, ccw_send, ccw_recv):
    my = lax.axis_index("i")
    left = lax.rem(my + N_DEV - 1, N_DEV)
    right = lax.rem(my + 1, N_DEV)
    j = pl.program_id(0)

    barrier_sem = pltpu.get_barrier_semaphore()
    for nbr in (left, right):
        pl.semaphore_signal(
            barrier_sem, inc=1,
            device_id=(nbr,), device_id_type=pl.DeviceIdType.MESH,
        )
    pl.semaphore_wait(barrier_sem, 2)

    def p_cw(c):
        return p_ref[pl.ds(c * M_BLK, M_BLK), :H]

    def p_ccw(c):
        return p_ref[pl.ds(c * M_BLK, M_BLK), H:]

    cw_ref[0] = p_cw(lax.rem(my + N_DEV - 1, N_DEV))
    ccw_ref[0] = p_ccw(lax.rem(my + 1, N_DEV))

    for h in range(N_DEV - 1):
        send_slot = h % 2
        recv_slot = (h + 1) % 2
        cw = pltpu.make_async_remote_copy(
            src_ref=cw_ref.at[send_slot],
            dst_ref=cw_ref.at[recv_slot],
            send_sem=cw_send.at[send_slot],
            recv_sem=cw_recv.at[recv_slot],
            device_id=(right,),
            device_id_type=pl.DeviceIdType.MESH,
        )
        ccw = pltpu.make_async_remote_copy(
            src_ref=ccw_ref.at[send_slot],
            dst_ref=ccw_ref.at[recv_slot],
            send_sem=ccw_send.at[send_slot],
            recv_sem=ccw_recv.at[recv_slot],
            device_id=(left,),
            device_id_type=pl.DeviceIdType.MESH,
        )
        cw.start()
        ccw.start()
        cw.wait()
        ccw.wait()

        c_cw = lax.rem(my + 2 * N_DEV - 2 - h, N_DEV)
        c_ccw = lax.rem(my + 2 + h, N_DEV)
        if h < N_DEV - 2:
            cw_ref[recv_slot] = cw_ref[recv_slot] + p_cw(c_cw)
            ccw_ref[recv_slot] = ccw_ref[recv_slot] + p_ccw(c_ccw)
        else:
            acc_cw = cw_ref[recv_slot].astype(jnp.float32) + p_cw(
                c_cw
            ).astype(jnp.float32)
            acc_ccw = ccw_ref[recv_slot].astype(jnp.float32) + p_ccw(
                c_ccw
            ).astype(jnp.float32)
            out_ref[:, :H] = acc_cw
            out_ref[:, H:] = acc_ccw
            local = jnp.maximum(jnp.max(acc_cw), jnp.max(acc_ccw))
            local_max = jnp.full((8, 128), jnp.maximum(local, 0.0))
            prev = jnp.where(j == 0, jnp.zeros((8, 128), jnp.float32),
                             amax_ref[...])
            amax_ref[...] = jnp.maximum(prev, local_max)


def _rs_call(partial):
    return pl.pallas_call(
        _rs_body,
        grid=(S,),
        in_specs=[
            pl.BlockSpec((N_DEV * M_BLK, W), lambda j: (0, j)),
        ],
        out_specs=[
            pl.BlockSpec((M_BLK, W), lambda j: (0, j)),
            pl.BlockSpec((8, 128), lambda j: (0, 0)),
        ],
        out_shape=[
            jax.ShapeDtypeStruct((M_BLK, N_TOT), jnp.float32),
            jax.ShapeDtypeStruct((8, 128), jnp.float32),
        ],
        scratch_shapes=[
            pltpu.VMEM((2, M_BLK, H), jnp.bfloat16),
            pltpu.VMEM((2, M_BLK, H), jnp.bfloat16),
            pltpu.SemaphoreType.DMA((2,)),
            pltpu.SemaphoreType.DMA((2,)),
            pltpu.SemaphoreType.DMA((2,)),
            pltpu.SemaphoreType.DMA((2,)),
        ],
        compiler_params=pltpu.CompilerParams(
            collective_id=0,
            dimension_semantics=("arbitrary",),
        ),
    )(partial)


def _amax_body(in_ref, out_ref, comm_ref, send_sems, recv_sems):
    my = lax.axis_index("i")

    barrier_sem = pltpu.get_barrier_semaphore()
    for k in range(1, N_DEV):
        pl.semaphore_signal(
            barrier_sem, inc=1,
            device_id=(lax.rem(my + k, N_DEV),),
            device_id_type=pl.DeviceIdType.MESH,
        )
    pl.semaphore_wait(barrier_sem, N_DEV - 1)

    rdmas = []
    for k in range(1, N_DEV):
        target = lax.rem(my + k, N_DEV)
        rdma = pltpu.make_async_remote_copy(
            src_ref=in_ref,
            dst_ref=comm_ref.at[k],
            send_sem=send_sems.at[k],
            recv_sem=recv_sems.at[k],
            device_id=(target,),
            device_id_type=pl.DeviceIdType.MESH,
        )
        rdma.start()
        rdmas.append(rdma)
    for rdma in rdmas:
        rdma.wait()

    result = in_ref[...]
    for k in range(1, N_DEV):
        result = jnp.maximum(result, comm_ref[k])
    out_ref[...] = result


def _amax_call(amax_local):
    return pl.pallas_call(
        _amax_body,
        in_specs=[pl.BlockSpec(memory_space=pltpu.VMEM)],
        out_specs=pl.BlockSpec(memory_space=pltpu.VMEM),
        out_shape=jax.ShapeDtypeStruct((8, 128), jnp.float32),
        scratch_shapes=[
            pltpu.VMEM((N_DEV, 8, 128), jnp.float32),
            pltpu.SemaphoreType.DMA((N_DEV,)),
            pltpu.SemaphoreType.DMA((N_DEV,)),
        ],
        compiler_params=pltpu.CompilerParams(collective_id=1),
    )(amax_local)


def kernel(x, w_mat):
    partial = jnp.dot(
        x, w_mat, preferred_element_type=jnp.float32
    ).astype(jnp.bfloat16)

    y, amax_local = _rs_call(partial)
    amax = _amax_call(amax_local)[0, 0]


    y = jnp.maximum(y, 0.0)
    scale = amax / 448.0
    q = (y / scale).astype(jnp.float8_e4m3fn)
    q = lax.optimization_barrier(q)
    return q.astype(jnp.float32) * scale


# device time: 399411 ns/iter; 1.7975x vs baseline; 1.1148x over previous
import jax
import jax.numpy as jnp
from jax import lax
from jax.experimental import pallas as pl
from jax.experimental.pallas import tpu as pltpu

N_DEV = 4
M_BLK = 1024
M_ALL = N_DEV * M_BLK
K_PER = 1024
N_TOT = 8192
W = 1024
S = N_TOT // W
H = W // 2


def _rs_body(x_hbm, w_hbm, out_ref, amax_ref,
             x_vmem, w_vmem, p_buf, cw_ref, ccw_ref,
             x_sem, w_sems, cw_send, cw_recv, ccw_send, ccw_recv):
    my = lax.axis_index("i")
    left = lax.rem(my + N_DEV - 1, N_DEV)
    right = lax.rem(my + 1, N_DEV)
    j = pl.program_id(0)
    cur = lax.rem(j, 2)
    nxt = lax.rem(j + 1, 2)

    barrier_sem = pltpu.get_barrier_semaphore()
    for nbr in (left, right):
        pl.semaphore_signal(
            barrier_sem, inc=1,
            device_id=(nbr,), device_id_type=pl.DeviceIdType.MESH,
        )
    pl.semaphore_wait(barrier_sem, 2)

    def gemm_into(slot, w_slot):
        for c in range(N_DEV):
            p_buf[pl.ds(slot * M_ALL + c * M_BLK, M_BLK), :] = jnp.dot(
                x_vmem[pl.ds(c * M_BLK, M_BLK), :],
                w_vmem[w_slot],
                preferred_element_type=jnp.float32,
            ).astype(jnp.bfloat16)

    @pl.when(j == 0)
    def _():
        xc = pltpu.make_async_copy(x_hbm, x_vmem, x_sem)
        xc.start()
        wc = pltpu.make_async_copy(
            w_hbm.at[:, pl.ds(0, W)], w_vmem.at[0], w_sems.at[0]
        )
        wc.start()
        xc.wait()
        wc.wait()
        gemm_into(0, 0)

    @pl.when(j + 1 < S)
    def _():
        pltpu.make_async_copy(
            w_hbm.at[:, pl.ds((j + 1) * W, W)], w_vmem.at[nxt],
            w_sems.at[nxt],
        ).start()

    def p_cw(c):
        return p_buf[pl.ds(cur * M_ALL + c * M_BLK, M_BLK), :H]

    def p_ccw(c):
        return p_buf[pl.ds(cur * M_ALL + c * M_BLK, M_BLK), H:]

    cw_ref[0] = p_cw(lax.rem(my + N_DEV - 1, N_DEV))
    ccw_ref[0] = p_ccw(lax.rem(my + 1, N_DEV))

    for h in range(N_DEV - 1):
        send_slot = h % 2
        recv_slot = (h + 1) % 2
        cw = pltpu.make_async_remote_copy(
            src_ref=cw_ref.at[send_slot],
            dst_ref=cw_ref.at[recv_slot],
            send_sem=cw_send.at[send_slot],
            recv_sem=cw_recv.at[recv_slot],
            device_id=(right,),
            device_id_type=pl.DeviceIdType.MESH,
        )
        ccw = pltpu.make_async_remote_copy(
            src_ref=ccw_ref.at[send_slot],
            dst_ref=ccw_ref.at[recv_slot],
            send_sem=ccw_send.at[send_slot],
            recv_sem=ccw_recv.at[recv_slot],
            device_id=(left,),
            device_id_type=pl.DeviceIdType.MESH,
        )
        cw.start()
        ccw.start()
        if h == 0:
            @pl.when(j + 1 < S)
            def _():
                pltpu.make_async_copy(
                    w_hbm.at[:, pl.ds((j + 1) * W, W)], w_vmem.at[nxt],
                    w_sems.at[nxt],
                ).wait()

                @pl.when(nxt == 1)
                def _():
                    gemm_into(1, 1)

                @pl.when(nxt == 0)
                def _():
                    gemm_into(0, 0)
        cw.wait()
        ccw.wait()

        c_cw = lax.rem(my + 2 * N_DEV - 2 - h, N_DEV)
        c_ccw = lax.rem(my + 2 + h, N_DEV)
        if h < N_DEV - 2:
            cw_ref[recv_slot] = cw_ref[recv_slot] + p_cw(c_cw)
            ccw_ref[recv_slot] = ccw_ref[recv_slot] + p_ccw(c_ccw)
        else:
            acc_cw = cw_ref[recv_slot].astype(jnp.float32) + p_cw(
                c_cw
            ).astype(jnp.float32)
            acc_ccw = ccw_ref[recv_slot].astype(jnp.float32) + p_ccw(
                c_ccw
            ).astype(jnp.float32)
            out_ref[:, :H] = acc_cw
            out_ref[:, H:] = acc_ccw
            local = jnp.maximum(jnp.max(acc_cw), jnp.max(acc_ccw))
            local_max = jnp.full((8, 128), jnp.maximum(local, 0.0))
            prev = jnp.where(j == 0, jnp.zeros((8, 128), jnp.float32),
                             amax_ref[...])
            amax_ref[...] = jnp.maximum(prev, local_max)


def _rs_call(x, w_mat):
    return pl.pallas_call(
        _rs_body,
        grid=(S,),
        in_specs=[
            pl.BlockSpec(memory_space=pltpu.MemorySpace.HBM),
            pl.BlockSpec(memory_space=pltpu.MemorySpace.HBM),
        ],
        out_specs=[
            pl.BlockSpec((M_BLK, W), lambda j: (0, j)),
            pl.BlockSpec((8, 128), lambda j: (0, 0)),
        ],
        out_shape=[
            jax.ShapeDtypeStruct((M_BLK, N_TOT), jnp.float32),
            jax.ShapeDtypeStruct((8, 128), jnp.float32),
        ],
        scratch_shapes=[
            pltpu.VMEM((M_ALL, K_PER), jnp.bfloat16),
            pltpu.VMEM((2, K_PER, W), jnp.bfloat16),
            pltpu.VMEM((2 * M_ALL, W), jnp.bfloat16),
            pltpu.VMEM((2, M_BLK, H), jnp.bfloat16),
            pltpu.VMEM((2, M_BLK, H), jnp.bfloat16),
            pltpu.SemaphoreType.DMA,
            pltpu.SemaphoreType.DMA((2,)),
            pltpu.SemaphoreType.DMA((2,)),
            pltpu.SemaphoreType.DMA((2,)),
            pltpu.SemaphoreType.DMA((2,)),
            pltpu.SemaphoreType.DMA((2,)),
        ],
        compiler_params=pltpu.CompilerParams(
            collective_id=0,
            dimension_semantics=("arbitrary",),
            vmem_limit_bytes=100 * 1024 * 1024,
        ),
    )(x, w_mat)


def _amax_body(in_ref, out_ref, comm_ref, send_sems, recv_sems):
    my = lax.axis_index("i")

    barrier_sem = pltpu.get_barrier_semaphore()
    for k in range(1, N_DEV):
        pl.semaphore_signal(
            barrier_sem, inc=1,
            device_id=(lax.rem(my + k, N_DEV),),
            device_id_type=pl.DeviceIdType.MESH,
        )
    pl.semaphore_wait(barrier_sem, N_DEV - 1)

    rdmas = []
    for k in range(1, N_DEV):
        target = lax.rem(my + k, N_DEV)
        rdma = pltpu.make_async_remote_copy(
            src_ref=in_ref,
            dst_ref=comm_ref.at[k],
            send_sem=send_sems.at[k],
            recv_sem=recv_sems.at[k],
            device_id=(target,),
            device_id_type=pl.DeviceIdType.MESH,
        )
        rdma.start()
        rdmas.append(rdma)
    for rdma in rdmas:
        rdma.wait()

    result = in_ref[...]
    for k in range(1, N_DEV):
        result = jnp.maximum(result, comm_ref[k])
    out_ref[...] = result


def _amax_call(amax_local):
    return pl.pallas_call(
        _amax_body,
        in_specs=[pl.BlockSpec(memory_space=pltpu.VMEM)],
        out_specs=pl.BlockSpec(memory_space=pltpu.VMEM),
        out_shape=jax.ShapeDtypeStruct((8, 128), jnp.float32),
        scratch_shapes=[
            pltpu.VMEM((N_DEV, 8, 128), jnp.float32),
            pltpu.SemaphoreType.DMA((N_DEV,)),
            pltpu.SemaphoreType.DMA((N_DEV,)),
        ],
        compiler_params=pltpu.CompilerParams(collective_id=1),
    )(amax_local)


def kernel(x, w_mat):
    y, amax_local = _rs_call(
        x.astype(jnp.bfloat16), w_mat.astype(jnp.bfloat16)
    )
    amax = _amax_call(amax_local)[0, 0]

    y = jnp.maximum(y, 0.0)
    scale = amax / 448.0
    q = (y / scale).astype(jnp.float8_e4m3fn)
    q = lax.optimization_barrier(q)
    return q.astype(jnp.float32) * scale


# device time: 372818 ns/iter; 1.9258x vs baseline; 1.0713x over previous
import jax
import jax.numpy as jnp
from jax import lax
from jax.experimental import pallas as pl
from jax.experimental.pallas import tpu as pltpu

N_DEV = 4
M_BLK = 1024
M_ALL = N_DEV * M_BLK
K_PER = 1024
N_TOT = 8192
W = 1024
S = N_TOT // W
H = W // 2
Q = W // 4


def _rs_body(x_hbm, w_hbm, out_ref, amax_ref,
             x_vmem, w_vmem, p_buf,
             cwa_ref, cwb_ref, ccwa_ref, ccwb_ref,
             x_sem, w_sems,
             cwa_send, cwa_recv, cwb_send, cwb_recv,
             ccwa_send, ccwa_recv, ccwb_send, ccwb_recv):
    my = lax.axis_index("i")
    left = lax.rem(my + N_DEV - 1, N_DEV)
    right = lax.rem(my + 1, N_DEV)
    j = pl.program_id(0)
    cur = lax.rem(j, 2)
    nxt = lax.rem(j + 1, 2)

    barrier_sem = pltpu.get_barrier_semaphore()
    for nbr in (left, right):
        pl.semaphore_signal(
            barrier_sem, inc=1,
            device_id=(nbr,), device_id_type=pl.DeviceIdType.MESH,
        )
    pl.semaphore_wait(barrier_sem, 2)

    def gemm_into(slot, w_slot):
        for c in range(N_DEV):
            p_buf[pl.ds(slot * M_ALL + c * M_BLK, M_BLK), :] = jnp.dot(
                x_vmem[pl.ds(c * M_BLK, M_BLK), :],
                w_vmem[w_slot],
                preferred_element_type=jnp.float32,
            ).astype(jnp.bfloat16)

    @pl.when(j == 0)
    def _():
        xc = pltpu.make_async_copy(x_hbm, x_vmem, x_sem)
        xc.start()
        wc = pltpu.make_async_copy(
            w_hbm.at[:, pl.ds(0, W)], w_vmem.at[0], w_sems.at[0]
        )
        wc.start()
        xc.wait()
        wc.wait()
        gemm_into(0, 0)

    @pl.when(j + 1 < S)
    def _():
        pltpu.make_async_copy(
            w_hbm.at[:, pl.ds((j + 1) * W, W)], w_vmem.at[nxt],
            w_sems.at[nxt],
        ).start()

    flows = [
        dict(comm=cwa_ref, ss=cwa_send, rs=cwa_recv, dev=right, lo=0,
             sign=-1),
        dict(comm=ccwa_ref, ss=ccwa_send, rs=ccwa_recv, dev=left, lo=2 * Q,
             sign=1),
        dict(comm=cwb_ref, ss=cwb_send, rs=cwb_recv, dev=right, lo=Q,
             sign=-1),
        dict(comm=ccwb_ref, ss=ccwb_send, rs=ccwb_recv, dev=left, lo=3 * Q,
             sign=1),
    ]

    def p_q(c, f):
        lo = f["lo"]
        return p_buf[pl.ds(cur * M_ALL + c * M_BLK, M_BLK), lo:lo + Q]

    def rdma(f, h):
        return pltpu.make_async_remote_copy(
            src_ref=f["comm"].at[h % 2],
            dst_ref=f["comm"].at[(h + 1) % 2],
            send_sem=f["ss"].at[h % 2],
            recv_sem=f["rs"].at[(h + 1) % 2],
            device_id=(f["dev"],),
            device_id_type=pl.DeviceIdType.MESH,
        )

    def chunk(f, h, recv):
        off = (1 + h + (1 if recv else 0)) * f["sign"]
        return lax.rem(my + off + 2 * N_DEV, N_DEV)

    for f in flows:
        f["comm"][0] = p_q(chunk(f, 0, recv=False), f)
        rdma(f, 0).start()

    @pl.when(j + 1 < S)
    def _():
        pltpu.make_async_copy(
            w_hbm.at[:, pl.ds((j + 1) * W, W)], w_vmem.at[nxt],
            w_sems.at[nxt],
        ).wait()

        @pl.when(nxt == 1)
        def _():
            gemm_into(1, 1)

        @pl.when(nxt == 0)
        def _():
            gemm_into(0, 0)

    local = None
    for h in range(N_DEV - 1):
        recv_slot = (h + 1) % 2
        for f in flows:
            rdma(f, h).wait()
            c = chunk(f, h, recv=True)
            if h < N_DEV - 2:
                f["comm"][recv_slot] = f["comm"][recv_slot] + p_q(c, f)
                rdma(f, h + 1).start()
            else:
                acc = f["comm"][recv_slot].astype(jnp.float32) + p_q(
                    c, f
                ).astype(jnp.float32)
                lo = f["lo"]
                out_ref[:, lo:lo + Q] = acc
                m = jnp.max(acc)
                local = m if local is None else jnp.maximum(local, m)

    local_max = jnp.full((8, 128), jnp.maximum(local, 0.0))
    prev = jnp.where(j == 0, jnp.zeros((8, 128), jnp.float32),
                     amax_ref[...])
    amax_ref[...] = jnp.maximum(prev, local_max)


def _rs_call(x, w_mat):
    return pl.pallas_call(
        _rs_body,
        grid=(S,),
        in_specs=[
            pl.BlockSpec(memory_space=pltpu.MemorySpace.HBM),
            pl.BlockSpec(memory_space=pltpu.MemorySpace.HBM),
        ],
        out_specs=[
            pl.BlockSpec((M_BLK, W), lambda j: (0, j)),
            pl.BlockSpec((8, 128), lambda j: (0, 0)),
        ],
        out_shape=[
            jax.ShapeDtypeStruct((M_BLK, N_TOT), jnp.float32),
            jax.ShapeDtypeStruct((8, 128), jnp.float32),
        ],
        scratch_shapes=[
            pltpu.VMEM((M_ALL, K_PER), jnp.bfloat16),
            pltpu.VMEM((2, K_PER, W), jnp.bfloat16),
            pltpu.VMEM((2 * M_ALL, W), jnp.bfloat16),
            pltpu.VMEM((2, M_BLK, Q), jnp.bfloat16),
            pltpu.VMEM((2, M_BLK, Q), jnp.bfloat16),
            pltpu.VMEM((2, M_BLK, Q), jnp.bfloat16),
            pltpu.VMEM((2, M_BLK, Q), jnp.bfloat16),
            pltpu.SemaphoreType.DMA,
            pltpu.SemaphoreType.DMA((2,)),
            pltpu.SemaphoreType.DMA((2,)),
            pltpu.SemaphoreType.DMA((2,)),
            pltpu.SemaphoreType.DMA((2,)),
            pltpu.SemaphoreType.DMA((2,)),
            pltpu.SemaphoreType.DMA((2,)),
            pltpu.SemaphoreType.DMA((2,)),
            pltpu.SemaphoreType.DMA((2,)),
            pltpu.SemaphoreType.DMA((2,)),
        ],
        compiler_params=pltpu.CompilerParams(
            collective_id=0,
            dimension_semantics=("arbitrary",),
            vmem_limit_bytes=100 * 1024 * 1024,
        ),
    )(x, w_mat)


def _amax_body(in_ref, out_ref, comm_ref, send_sems, recv_sems):
    my = lax.axis_index("i")

    barrier_sem = pltpu.get_barrier_semaphore()
    for k in range(1, N_DEV):
        pl.semaphore_signal(
            barrier_sem, inc=1,
            device_id=(lax.rem(my + k, N_DEV),),
            device_id_type=pl.DeviceIdType.MESH,
        )
    pl.semaphore_wait(barrier_sem, N_DEV - 1)

    rdmas = []
    for k in range(1, N_DEV):
        target = lax.rem(my + k, N_DEV)
        rdma = pltpu.make_async_remote_copy(
            src_ref=in_ref,
            dst_ref=comm_ref.at[k],
            send_sem=send_sems.at[k],
            recv_sem=recv_sems.at[k],
            device_id=(target,),
            device_id_type=pl.DeviceIdType.MESH,
        )
        rdma.start()
        rdmas.append(rdma)
    for rdma in rdmas:
        rdma.wait()

    result = in_ref[...]
    for k in range(1, N_DEV):
        result = jnp.maximum(result, comm_ref[k])
    out_ref[...] = result


def _amax_call(amax_local):
    return pl.pallas_call(
        _amax_body,
        in_specs=[pl.BlockSpec(memory_space=pltpu.VMEM)],
        out_specs=pl.BlockSpec(memory_space=pltpu.VMEM),
        out_shape=jax.ShapeDtypeStruct((8, 128), jnp.float32),
        scratch_shapes=[
            pltpu.VMEM((N_DEV, 8, 128), jnp.float32),
            pltpu.SemaphoreType.DMA((N_DEV,)),
            pltpu.SemaphoreType.DMA((N_DEV,)),
        ],
        compiler_params=pltpu.CompilerParams(collective_id=1),
    )(amax_local)


def kernel(x, w_mat):
    y, amax_local = _rs_call(
        x.astype(jnp.bfloat16), w_mat.astype(jnp.bfloat16)
    )
    amax = _amax_call(amax_local)[0, 0]

    y = jnp.maximum(y, 0.0)
    scale = amax / 448.0
    q = (y / scale).astype(jnp.float8_e4m3fn)
    q = lax.optimization_barrier(q)
    return q.astype(jnp.float32) * scale
